# Initial kernel scaffold; baseline (speedup 1.0000x reference)
import jax
import jax.numpy as jnp
from jax.experimental import pallas as pl


def kernel(features, edge_index, proj_W, proj_b, gnn_W0, gnn_b0, gnn_W1, gnn_b1, W_ih, W_hh, b_ih, b_hh, fc_W, fc_b):
    raise NotImplementedError("write your pallas kernel here")


# speedup vs baseline: 7.4878x; 7.4878x over previous
import functools

import jax
import jax.numpy as jnp
from jax import lax
from jax.experimental import pallas as pl
from jax.experimental.pallas import tpu as pltpu
from jax.experimental.pallas import tpu_sc as plsc

N = 10000
E = 320000
H = 128
C = 16
G4 = 4 * H

NC = 2
NS = 16
NW = NC * NS

NP = 10240
RP = NP // NS
EPW = E // NW
CH = 128
NCHUNK = EPW // CH
TAIL = EPW - NCHUNK * CH

F32 = jnp.float32

_SC_MESH = dict(core_axis_name="c", subcore_axis_name="s",
                num_cores=NC, num_subcores=NS)

PREC = jax.lax.Precision.HIGHEST


def _deg_body(src_hbm, dst_hbm, ones_a_hbm, ones_b_hbm, z_hbm, out_hbm,
              sidx_v, didx_v, sidx_t, didx_t, ones_a_v, ones_b_v, acc_sh):
    c = lax.axis_index("c")
    s = lax.axis_index("s")
    w = s * NC + c
    rp0 = s * RP
    pltpu.sync_copy(z_hbm.at[pl.ds(rp0, RP), :], acc_sh.at[pl.ds(rp0, RP), :])
    pltpu.sync_copy(ones_a_hbm, ones_a_v)
    pltpu.sync_copy(ones_b_hbm, ones_b_v)
    plsc.subcore_barrier()

    base = w * EPW

    def body(ci, carry):
        off = base + ci * CH
        pltpu.sync_copy(src_hbm.at[pl.ds(off, CH)], sidx_v)
        pltpu.sync_copy(ones_a_v, acc_sh.at[sidx_v], add=True)
        pltpu.sync_copy(dst_hbm.at[pl.ds(off, CH)], didx_v)
        pltpu.sync_copy(ones_b_v, acc_sh.at[didx_v], add=True)
        return carry

    lax.fori_loop(0, NCHUNK, body, 0)

    offt = base + NCHUNK * CH
    pltpu.sync_copy(src_hbm.at[pl.ds(offt, TAIL)], sidx_t)
    pltpu.sync_copy(ones_a_v.at[pl.ds(0, TAIL), :], acc_sh.at[sidx_t], add=True)
    pltpu.sync_copy(dst_hbm.at[pl.ds(offt, TAIL)], didx_t)
    pltpu.sync_copy(ones_b_v.at[pl.ds(0, TAIL), :], acc_sh.at[didx_t], add=True)

    plsc.subcore_barrier()
    pltpu.sync_copy(acc_sh.at[pl.ds(rp0, RP), :], out_hbm.at[c, pl.ds(rp0, RP), :])


@functools.lru_cache(maxsize=None)
def _get_deg_call():
    return pl.kernel(
        _deg_body,
        out_type=jax.ShapeDtypeStruct((NC, NP, H), F32),
        mesh=plsc.VectorSubcoreMesh(**_SC_MESH),
        scratch_types=[
            pltpu.VMEM((CH,), jnp.int32),
            pltpu.VMEM((CH,), jnp.int32),
            pltpu.VMEM((TAIL,), jnp.int32),
            pltpu.VMEM((TAIL,), jnp.int32),
            pltpu.VMEM((CH, H), F32),
            pltpu.VMEM((CH, H), F32),
            pltpu.VMEM_SHARED((NP, H), F32),
        ],
    )


def _agg_body(hs_hbm, src_hbm, dst_hbm, z_hbm, out_hbm,
              sidx_v, didx_v, sidx_t, didx_t, rows_v, acc_sh, sem):
    c = lax.axis_index("c")
    s = lax.axis_index("s")
    w = s * NC + c
    rp0 = s * RP
    pltpu.sync_copy(z_hbm.at[pl.ds(rp0, RP), :], acc_sh.at[pl.ds(rp0, RP), :])
    plsc.subcore_barrier()

    base = w * EPW

    def body(ci, carry):
        off = base + ci * CH
        pltpu.sync_copy(src_hbm.at[pl.ds(off, CH)], sidx_v)
        pltpu.async_copy(hs_hbm.at[sidx_v], rows_v, sem).wait()
        pltpu.sync_copy(dst_hbm.at[pl.ds(off, CH)], didx_v)
        pltpu.sync_copy(rows_v, acc_sh.at[didx_v], add=True)
        return carry

    lax.fori_loop(0, NCHUNK, body, 0)

    offt = base + NCHUNK * CH
    pltpu.sync_copy(src_hbm.at[pl.ds(offt, TAIL)], sidx_t)
    pltpu.async_copy(hs_hbm.at[sidx_t], rows_v.at[pl.ds(0, TAIL), :], sem).wait()
    pltpu.sync_copy(dst_hbm.at[pl.ds(offt, TAIL)], didx_t)
    pltpu.sync_copy(rows_v.at[pl.ds(0, TAIL), :], acc_sh.at[didx_t], add=True)

    plsc.subcore_barrier()
    pltpu.sync_copy(acc_sh.at[pl.ds(rp0, RP), :], out_hbm.at[c, pl.ds(rp0, RP), :])


@functools.lru_cache(maxsize=None)
def _get_agg_call():
    return pl.kernel(
        _agg_body,
        out_type=jax.ShapeDtypeStruct((NC, NP, H), F32),
        mesh=plsc.VectorSubcoreMesh(**_SC_MESH),
        scratch_types=[
            pltpu.VMEM((CH,), jnp.int32),
            pltpu.VMEM((CH,), jnp.int32),
            pltpu.VMEM((TAIL,), jnp.int32),
            pltpu.VMEM((TAIL,), jnp.int32),
            pltpu.VMEM((CH, H), F32),
            pltpu.VMEM_SHARED((NP, H), F32),
            pltpu.SemaphoreType.DMA,
        ],
    )


BR = 2000


DEG_OUT_COL = 0
DEG_IN_COL = 64


def _norm_col(d0, d1, col):
    deg = d0[:, col:col + 1] + d1[:, col:col + 1]
    return jax.lax.rsqrt(jnp.maximum(deg, 1.0))


def _proj_body(x_ref, w_ref, b_ref, d0_ref, d1_ref, o_ref):
    nout = _norm_col(d0_ref[...], d1_ref[...], DEG_OUT_COL)
    h = jnp.dot(x_ref[...], w_ref[...], preferred_element_type=F32,
                precision=PREC) + b_ref[...]
    o_ref[...] = h * nout


def _conv_mid_body(a0_ref, a1_ref, d0_ref, d1_ref, w_ref, b_ref, o_ref):
    nin = _norm_col(d0_ref[...], d1_ref[...], DEG_IN_COL)
    agg = (a0_ref[...] + a1_ref[...]) * nin
    h = jnp.dot(agg, w_ref[...], preferred_element_type=F32,
                precision=PREC) + b_ref[...]
    h = jnp.maximum(h, 0.0)
    nout = _norm_col(d0_ref[...], d1_ref[...], DEG_OUT_COL)
    o_ref[...] = h * nout


def _conv_last_body(a0_ref, a1_ref, d0_ref, d1_ref, w_ref, b_ref, o_ref):
    nin = _norm_col(d0_ref[...], d1_ref[...], DEG_IN_COL)
    agg = (a0_ref[...] + a1_ref[...]) * nin
    h = jnp.dot(agg, w_ref[...], preferred_element_type=F32,
                precision=PREC) + b_ref[...]
    o_ref[...] = jnp.maximum(h, 0.0)


def _row_spec(width):
    return pl.BlockSpec((BR, width), lambda i: (i, 0))


def _full_spec(shape):
    return pl.BlockSpec(shape, lambda i: tuple(0 for _ in shape))


def _proj_call(x, w, b, d0, d1):
    return pl.pallas_call(
        _proj_body,
        grid=(N // BR,),
        in_specs=[_row_spec(H), _full_spec((H, H)), _full_spec((1, H)),
                  _row_spec(H), _row_spec(H)],
        out_specs=_row_spec(H),
        out_shape=jax.ShapeDtypeStruct((N, H), F32),
    )(x, w, b, d0, d1)


def _conv_mid_call(a0, a1, d0, d1, w, b):
    return pl.pallas_call(
        _conv_mid_body,
        grid=(N // BR,),
        in_specs=[_row_spec(H), _row_spec(H), _row_spec(H), _row_spec(H),
                  _full_spec((H, H)), _full_spec((1, H))],
        out_specs=_row_spec(H),
        out_shape=jax.ShapeDtypeStruct((N, H), F32),
    )(a0, a1, d0, d1, w, b)


def _conv_last_call(a0, a1, d0, d1, w, b):
    return pl.pallas_call(
        _conv_last_body,
        grid=(N // BR,),
        in_specs=[_row_spec(H), _row_spec(H), _row_spec(H), _row_spec(H),
                  _full_spec((H, H)), _full_spec((1, H))],
        out_specs=_row_spec(H),
        out_shape=jax.ShapeDtypeStruct((N, H), F32),
    )(a0, a1, d0, d1, w, b)


TB = 1000


def _sigmoid(z):
    return 0.5 * jnp.tanh(0.5 * z) + 0.5


def _lstm_body(h2_ref, wih_ref, bih_ref, whh_ref, fcw_ref, fcb_ref, o_ref,
               h_st, c_st, x_st, hblk_st):
    @pl.when(pl.program_id(0) == 0)
    def _init():
        h_st[...] = jnp.zeros_like(h_st)
        c_st[...] = jnp.zeros_like(c_st)

    x_st[...] = jnp.dot(h2_ref[...], wih_ref[...], preferred_element_type=F32,
                        precision=PREC) + bih_ref[...]
    wh = whh_ref[...]

    def step(t, carry):
        h, c = carry
        x = x_st[pl.ds(t, 1), :]
        gates = x + jnp.dot(h, wh, preferred_element_type=F32, precision=PREC)
        i = _sigmoid(gates[:, 0:H])
        f = _sigmoid(gates[:, H:2 * H])
        g = jnp.tanh(gates[:, 2 * H:3 * H])
        o = _sigmoid(gates[:, 3 * H:4 * H])
        c = f * c + i * g
        h = o * jnp.tanh(c)
        hblk_st[pl.ds(t, 1), :] = h
        return (h, c)

    hn, cn = lax.fori_loop(0, TB, step, (h_st[...], c_st[...]))
    h_st[...] = hn
    c_st[...] = cn
    o_ref[...] = jnp.dot(hblk_st[...], fcw_ref[...], preferred_element_type=F32,
                         precision=PREC) + fcb_ref[...]


def _lstm_call(h2, wih_t, bias, whh_t, fcw, fcb):
    return pl.pallas_call(
        _lstm_body,
        grid=(N // TB,),
        in_specs=[pl.BlockSpec((TB, H), lambda i: (i, 0)),
                  _full_spec((H, G4)), _full_spec((1, G4)),
                  _full_spec((H, G4)), _full_spec((H, C)), _full_spec((1, C))],
        out_specs=pl.BlockSpec((TB, C), lambda i: (i, 0)),
        out_shape=jax.ShapeDtypeStruct((N, C), F32),
        scratch_shapes=[
            pltpu.VMEM((1, H), F32),
            pltpu.VMEM((1, H), F32),
            pltpu.VMEM((TB, G4), F32),
            pltpu.VMEM((TB, H), F32),
        ],
        compiler_params=pltpu.CompilerParams(
            dimension_semantics=("arbitrary",)),
    )(h2, wih_t, bias, whh_t, fcw, fcb)


def kernel(features, edge_index, proj_W, proj_b, gnn_W0, gnn_b0, gnn_W1,
           gnn_b1, W_ih, W_hh, b_ih, b_hh, fc_W, fc_b):
    src = edge_index[0]
    dst = edge_index[1]
    z128 = jnp.zeros((NP, H), F32)
    col = jnp.arange(H)
    ones_a = jnp.broadcast_to((col < 64).astype(F32), (CH, H))
    ones_b = jnp.broadcast_to((col >= 64).astype(F32), (CH, H))

    deg_p = _get_deg_call()(src, dst, ones_a, ones_b, z128)
    d0, d1 = deg_p[0], deg_p[1]

    hs0 = _proj_call(features, proj_W, proj_b.reshape(1, H), d0, d1)
    agg0 = _get_agg_call()(hs0, src, dst, z128)
    hs1 = _conv_mid_call(agg0[0], agg0[1], d0, d1, gnn_W0,
                         gnn_b0.reshape(1, H))
    agg1 = _get_agg_call()(hs1, src, dst, z128)
    h2 = _conv_last_call(agg1[0], agg1[1], d0, d1, gnn_W1,
                         gnn_b1.reshape(1, H))

    out = _lstm_call(h2, W_ih.T, (b_ih + b_hh).reshape(1, G4), W_hh.T,
                     fc_W, fc_b.reshape(1, C))
    return out


# speedup vs baseline: 11.6755x; 1.5593x over previous
import functools

import jax
import jax.numpy as jnp
from jax import lax
from jax.experimental import pallas as pl
from jax.experimental.pallas import tpu as pltpu
from jax.experimental.pallas import tpu_sc as plsc

N = 10000
E = 320000
H = 128
C = 16
G4 = 4 * H

NC = 2
NS = 16
NW = NC * NS

NP = 10240
RP = NP // NS
EPW = E // NW
CH = 128
NCHUNK = EPW // CH
TAIL = EPW - NCHUNK * CH

F32 = jnp.float32

_SC_MESH = dict(core_axis_name="c", subcore_axis_name="s",
                num_cores=NC, num_subcores=NS)

PREC = jax.lax.Precision.HIGHEST
PREC_STEP = jax.lax.Precision.DEFAULT


def _deg_body(src_hbm, dst_hbm, ones_a_hbm, ones_b_hbm, z_hbm, out_hbm,
              sidx_v, didx_v, sidx_t, didx_t, ones_a_v, ones_b_v, acc_sh):
    c = lax.axis_index("c")
    s = lax.axis_index("s")
    w = s * NC + c
    rp0 = s * RP
    pltpu.sync_copy(z_hbm.at[pl.ds(rp0, RP), :], acc_sh.at[pl.ds(rp0, RP), :])
    pltpu.sync_copy(ones_a_hbm, ones_a_v)
    pltpu.sync_copy(ones_b_hbm, ones_b_v)
    plsc.subcore_barrier()

    base = w * EPW

    def body(ci, carry):
        off = base + ci * CH
        pltpu.sync_copy(src_hbm.at[pl.ds(off, CH)], sidx_v)
        pltpu.sync_copy(ones_a_v, acc_sh.at[sidx_v], add=True)
        pltpu.sync_copy(dst_hbm.at[pl.ds(off, CH)], didx_v)
        pltpu.sync_copy(ones_b_v, acc_sh.at[didx_v], add=True)
        return carry

    lax.fori_loop(0, NCHUNK, body, 0)

    offt = base + NCHUNK * CH
    pltpu.sync_copy(src_hbm.at[pl.ds(offt, TAIL)], sidx_t)
    pltpu.sync_copy(ones_a_v.at[pl.ds(0, TAIL), :], acc_sh.at[sidx_t], add=True)
    pltpu.sync_copy(dst_hbm.at[pl.ds(offt, TAIL)], didx_t)
    pltpu.sync_copy(ones_b_v.at[pl.ds(0, TAIL), :], acc_sh.at[didx_t], add=True)

    plsc.subcore_barrier()
    pltpu.sync_copy(acc_sh.at[pl.ds(rp0, RP), :], out_hbm.at[c, pl.ds(rp0, RP), :])


@functools.lru_cache(maxsize=None)
def _get_deg_call():
    return pl.kernel(
        _deg_body,
        out_type=jax.ShapeDtypeStruct((NC, NP, H), F32),
        mesh=plsc.VectorSubcoreMesh(**_SC_MESH),
        scratch_types=[
            pltpu.VMEM((CH,), jnp.int32),
            pltpu.VMEM((CH,), jnp.int32),
            pltpu.VMEM((TAIL,), jnp.int32),
            pltpu.VMEM((TAIL,), jnp.int32),
            pltpu.VMEM((CH, H), F32),
            pltpu.VMEM((CH, H), F32),
            pltpu.VMEM_SHARED((NP, H), F32),
        ],
    )


def _agg_body(hs_hbm, src_hbm, dst_hbm, z_hbm, out_hbm,
              sidx_v, didx_v, sidx_t, didx_t, rows_v, acc_sh, sem):
    c = lax.axis_index("c")
    s = lax.axis_index("s")
    w = s * NC + c
    rp0 = s * RP
    pltpu.sync_copy(z_hbm.at[pl.ds(rp0, RP), :], acc_sh.at[pl.ds(rp0, RP), :])
    plsc.subcore_barrier()

    base = w * EPW

    def body(ci, carry):
        off = base + ci * CH
        pltpu.sync_copy(src_hbm.at[pl.ds(off, CH)], sidx_v)
        pltpu.async_copy(hs_hbm.at[sidx_v], rows_v, sem).wait()
        pltpu.sync_copy(dst_hbm.at[pl.ds(off, CH)], didx_v)
        pltpu.sync_copy(rows_v, acc_sh.at[didx_v], add=True)
        return carry

    lax.fori_loop(0, NCHUNK, body, 0)

    offt = base + NCHUNK * CH
    pltpu.sync_copy(src_hbm.at[pl.ds(offt, TAIL)], sidx_t)
    pltpu.async_copy(hs_hbm.at[sidx_t], rows_v.at[pl.ds(0, TAIL), :], sem).wait()
    pltpu.sync_copy(dst_hbm.at[pl.ds(offt, TAIL)], didx_t)
    pltpu.sync_copy(rows_v.at[pl.ds(0, TAIL), :], acc_sh.at[didx_t], add=True)

    plsc.subcore_barrier()
    pltpu.sync_copy(acc_sh.at[pl.ds(rp0, RP), :], out_hbm.at[c, pl.ds(rp0, RP), :])


@functools.lru_cache(maxsize=None)
def _get_agg_call():
    return pl.kernel(
        _agg_body,
        out_type=jax.ShapeDtypeStruct((NC, NP, H), F32),
        mesh=plsc.VectorSubcoreMesh(**_SC_MESH),
        scratch_types=[
            pltpu.VMEM((CH,), jnp.int32),
            pltpu.VMEM((CH,), jnp.int32),
            pltpu.VMEM((TAIL,), jnp.int32),
            pltpu.VMEM((TAIL,), jnp.int32),
            pltpu.VMEM((CH, H), F32),
            pltpu.VMEM_SHARED((NP, H), F32),
            pltpu.SemaphoreType.DMA,
        ],
    )


BR = 2000


DEG_OUT_COL = 0
DEG_IN_COL = 64


def _norm_col(d0, d1, col):
    deg = d0[:, col:col + 1] + d1[:, col:col + 1]
    return jax.lax.rsqrt(jnp.maximum(deg, 1.0))


def _proj_body(x_ref, w_ref, b_ref, d0_ref, d1_ref, o_ref):
    nout = _norm_col(d0_ref[...], d1_ref[...], DEG_OUT_COL)
    h = jnp.dot(x_ref[...], w_ref[...], preferred_element_type=F32,
                precision=PREC) + b_ref[...]
    o_ref[...] = h * nout


def _conv_mid_body(a0_ref, a1_ref, d0_ref, d1_ref, w_ref, b_ref, o_ref):
    nin = _norm_col(d0_ref[...], d1_ref[...], DEG_IN_COL)
    agg = (a0_ref[...] + a1_ref[...]) * nin
    h = jnp.dot(agg, w_ref[...], preferred_element_type=F32,
                precision=PREC) + b_ref[...]
    h = jnp.maximum(h, 0.0)
    nout = _norm_col(d0_ref[...], d1_ref[...], DEG_OUT_COL)
    o_ref[...] = h * nout


def _conv_last_body(a0_ref, a1_ref, d0_ref, d1_ref, w_ref, b_ref, o_ref):
    nin = _norm_col(d0_ref[...], d1_ref[...], DEG_IN_COL)
    agg = (a0_ref[...] + a1_ref[...]) * nin
    h = jnp.dot(agg, w_ref[...], preferred_element_type=F32,
                precision=PREC) + b_ref[...]
    o_ref[...] = jnp.maximum(h, 0.0)


def _row_spec(width):
    return pl.BlockSpec((BR, width), lambda i: (i, 0))


def _full_spec(shape):
    return pl.BlockSpec(shape, lambda i: tuple(0 for _ in shape))


def _proj_call(x, w, b, d0, d1):
    return pl.pallas_call(
        _proj_body,
        grid=(N // BR,),
        in_specs=[_row_spec(H), _full_spec((H, H)), _full_spec((1, H)),
                  _row_spec(H), _row_spec(H)],
        out_specs=_row_spec(H),
        out_shape=jax.ShapeDtypeStruct((N, H), F32),
    )(x, w, b, d0, d1)


def _conv_mid_call(a0, a1, d0, d1, w, b):
    return pl.pallas_call(
        _conv_mid_body,
        grid=(N // BR,),
        in_specs=[_row_spec(H), _row_spec(H), _row_spec(H), _row_spec(H),
                  _full_spec((H, H)), _full_spec((1, H))],
        out_specs=_row_spec(H),
        out_shape=jax.ShapeDtypeStruct((N, H), F32),
    )(a0, a1, d0, d1, w, b)


def _conv_last_call(a0, a1, d0, d1, w, b):
    return pl.pallas_call(
        _conv_last_body,
        grid=(N // BR,),
        in_specs=[_row_spec(H), _row_spec(H), _row_spec(H), _row_spec(H),
                  _full_spec((H, H)), _full_spec((1, H))],
        out_specs=_row_spec(H),
        out_shape=jax.ShapeDtypeStruct((N, H), F32),
    )(a0, a1, d0, d1, w, b)


TB = 1000


def _sigmoid(z):
    return 0.5 * jnp.tanh(0.5 * z) + 0.5


def _lstm_body(h2_ref, wih_ref, bih_ref, whh_ref, fcw_ref, fcb_ref, o_ref,
               h_st, c_st, x_st, hblk_st):
    @pl.when(pl.program_id(0) == 0)
    def _init():
        h_st[...] = jnp.zeros_like(h_st)
        c_st[...] = jnp.zeros_like(c_st)

    x_st[...] = jnp.dot(h2_ref[...], wih_ref[...], preferred_element_type=F32,
                        precision=PREC) + bih_ref[...]
    wh = whh_ref[...]

    def step(t, carry):
        h, c = carry
        x = x_st[pl.ds(t, 1), :]
        gates = x + jnp.dot(h.astype(jnp.bfloat16), wh,
                            preferred_element_type=F32, precision=PREC_STEP)
        i = _sigmoid(gates[:, 0:H])
        f = _sigmoid(gates[:, H:2 * H])
        g = jnp.tanh(gates[:, 2 * H:3 * H])
        o = _sigmoid(gates[:, 3 * H:4 * H])
        c = f * c + i * g
        h = o * jnp.tanh(c)
        hblk_st[pl.ds(t, 1), :] = h
        return (h, c)

    hn, cn = lax.fori_loop(0, TB, step, (h_st[...], c_st[...]), unroll=2)
    h_st[...] = hn
    c_st[...] = cn
    o_ref[...] = jnp.dot(hblk_st[...], fcw_ref[...], preferred_element_type=F32,
                         precision=PREC) + fcb_ref[...]


def _lstm_call(h2, wih_t, bias, whh_t, fcw, fcb):
    return pl.pallas_call(
        _lstm_body,
        grid=(N // TB,),
        in_specs=[pl.BlockSpec((TB, H), lambda i: (i, 0)),
                  _full_spec((H, G4)), _full_spec((1, G4)),
                  _full_spec((H, G4)), _full_spec((H, C)), _full_spec((1, C))],
        out_specs=pl.BlockSpec((TB, C), lambda i: (i, 0)),
        out_shape=jax.ShapeDtypeStruct((N, C), F32),
        scratch_shapes=[
            pltpu.VMEM((1, H), F32),
            pltpu.VMEM((1, H), F32),
            pltpu.VMEM((TB, G4), F32),
            pltpu.VMEM((TB, H), F32),
        ],
        compiler_params=pltpu.CompilerParams(
            dimension_semantics=("arbitrary",)),
    )(h2, wih_t, bias, whh_t, fcw, fcb)


def kernel(features, edge_index, proj_W, proj_b, gnn_W0, gnn_b0, gnn_W1,
           gnn_b1, W_ih, W_hh, b_ih, b_hh, fc_W, fc_b):
    src = edge_index[0]
    dst = edge_index[1]
    z128 = jnp.zeros((NP, H), F32)
    col = jnp.arange(H)
    ones_a = jnp.broadcast_to((col < 64).astype(F32), (CH, H))
    ones_b = jnp.broadcast_to((col >= 64).astype(F32), (CH, H))

    deg_p = _get_deg_call()(src, dst, ones_a, ones_b, z128)
    d0, d1 = deg_p[0], deg_p[1]

    hs0 = _proj_call(features, proj_W, proj_b.reshape(1, H), d0, d1)
    agg0 = _get_agg_call()(hs0, src, dst, z128)
    hs1 = _conv_mid_call(agg0[0], agg0[1], d0, d1, gnn_W0,
                         gnn_b0.reshape(1, H))
    agg1 = _get_agg_call()(hs1, src, dst, z128)
    h2 = _conv_last_call(agg1[0], agg1[1], d0, d1, gnn_W1,
                         gnn_b1.reshape(1, H))

    out = _lstm_call(h2, W_ih.T, (b_ih + b_hh).reshape(1, G4),
                     W_hh.T.astype(jnp.bfloat16), fc_W, fc_b.reshape(1, C))
    return out


# speedup vs baseline: 12.7746x; 1.0941x over previous
import functools

import jax
import jax.numpy as jnp
from jax import lax
from jax.experimental import pallas as pl
from jax.experimental.pallas import tpu as pltpu
from jax.experimental.pallas import tpu_sc as plsc

N = 10000
E = 320000
H = 128
C = 16
G4 = 4 * H

NC = 2
NS = 16
NW = NC * NS

NP = 10240
RP = NP // NS
EPW = E // NW
CH = 128
NCHUNK = EPW // CH
TAIL = EPW - NCHUNK * CH

F32 = jnp.float32

_SC_MESH = dict(core_axis_name="c", subcore_axis_name="s",
                num_cores=NC, num_subcores=NS)

PREC = jax.lax.Precision.HIGHEST
PREC_STEP = jax.lax.Precision.DEFAULT


def _deg_body(src_hbm, dst_hbm, ones_a_hbm, ones_b_hbm, z_hbm, out_hbm,
              sidx_v, didx_v, sidx_t, didx_t, ones_a_v, ones_b_v, acc_sh):
    c = lax.axis_index("c")
    s = lax.axis_index("s")
    w = s * NC + c
    rp0 = s * RP
    pltpu.sync_copy(z_hbm.at[pl.ds(rp0, RP), :], acc_sh.at[pl.ds(rp0, RP), :])
    pltpu.sync_copy(ones_a_hbm, ones_a_v)
    pltpu.sync_copy(ones_b_hbm, ones_b_v)
    plsc.subcore_barrier()

    base = w * EPW

    def body(ci, carry):
        off = base + ci * CH
        pltpu.sync_copy(src_hbm.at[pl.ds(off, CH)], sidx_v)
        pltpu.sync_copy(ones_a_v, acc_sh.at[sidx_v], add=True)
        pltpu.sync_copy(dst_hbm.at[pl.ds(off, CH)], didx_v)
        pltpu.sync_copy(ones_b_v, acc_sh.at[didx_v], add=True)
        return carry

    lax.fori_loop(0, NCHUNK, body, 0)

    offt = base + NCHUNK * CH
    pltpu.sync_copy(src_hbm.at[pl.ds(offt, TAIL)], sidx_t)
    pltpu.sync_copy(ones_a_v.at[pl.ds(0, TAIL), :], acc_sh.at[sidx_t], add=True)
    pltpu.sync_copy(dst_hbm.at[pl.ds(offt, TAIL)], didx_t)
    pltpu.sync_copy(ones_b_v.at[pl.ds(0, TAIL), :], acc_sh.at[didx_t], add=True)

    plsc.subcore_barrier()
    pltpu.sync_copy(acc_sh.at[pl.ds(rp0, RP), :], out_hbm.at[c, pl.ds(rp0, RP), :])


@functools.lru_cache(maxsize=None)
def _get_deg_call():
    return pl.kernel(
        _deg_body,
        out_type=jax.ShapeDtypeStruct((NC, NP, H), F32),
        mesh=plsc.VectorSubcoreMesh(**_SC_MESH),
        scratch_types=[
            pltpu.VMEM((CH,), jnp.int32),
            pltpu.VMEM((CH,), jnp.int32),
            pltpu.VMEM((TAIL,), jnp.int32),
            pltpu.VMEM((TAIL,), jnp.int32),
            pltpu.VMEM((CH, H), F32),
            pltpu.VMEM((CH, H), F32),
            pltpu.VMEM_SHARED((NP, H), F32),
        ],
    )


def _agg_body(hs_hbm, src_hbm, dst_hbm, z_hbm, out_hbm,
              sidx0, sidx1, didx0, didx1, sidx_t, didx_t,
              rows0, rows1, rows_t, acc_sh, sem0, sem1):
    c = lax.axis_index("c")
    s = lax.axis_index("s")
    w = s * NC + c
    rp0 = s * RP
    pltpu.sync_copy(z_hbm.at[pl.ds(rp0, RP), :], acc_sh.at[pl.ds(rp0, RP), :])
    plsc.subcore_barrier()

    base = w * EPW

    def gather(idx_ref, rows_ref, sem, off):
        pltpu.sync_copy(src_hbm.at[pl.ds(off, CH)], idx_ref)
        pltpu.async_copy(hs_hbm.at[idx_ref], rows_ref, sem)

    def drain_scatter(idx_ref, rows_ref, sem, off):
        pltpu.make_async_copy(hs_hbm.at[idx_ref], rows_ref, sem).wait()
        pltpu.sync_copy(dst_hbm.at[pl.ds(off, CH)],
                        didx0 if rows_ref is rows0 else didx1)
        pltpu.sync_copy(rows_ref,
                        acc_sh.at[didx0 if rows_ref is rows0 else didx1],
                        add=True)

    gather(sidx0, rows0, sem0, base)

    def pair(k, carry):
        off0 = base + (2 * k) * CH
        gather(sidx1, rows1, sem1, off0 + CH)
        drain_scatter(sidx0, rows0, sem0, off0)
        gather(sidx0, rows0, sem0, off0 + 2 * CH)
        drain_scatter(sidx1, rows1, sem1, off0 + CH)
        return carry

    lax.fori_loop(0, (NCHUNK - 2) // 2, pair, 0)

    off76 = base + (NCHUNK - 2) * CH
    gather(sidx1, rows1, sem1, off76 + CH)
    drain_scatter(sidx0, rows0, sem0, off76)
    offt = base + NCHUNK * CH
    pltpu.sync_copy(src_hbm.at[pl.ds(offt, TAIL)], sidx_t)
    pltpu.async_copy(hs_hbm.at[sidx_t], rows_t, sem0)
    drain_scatter(sidx1, rows1, sem1, off76 + CH)
    pltpu.make_async_copy(hs_hbm.at[sidx_t], rows_t, sem0).wait()
    pltpu.sync_copy(dst_hbm.at[pl.ds(offt, TAIL)], didx_t)
    pltpu.sync_copy(rows_t, acc_sh.at[didx_t], add=True)

    plsc.subcore_barrier()
    pltpu.sync_copy(acc_sh.at[pl.ds(rp0, RP), :], out_hbm.at[c, pl.ds(rp0, RP), :])


@functools.lru_cache(maxsize=None)
def _get_agg_call():
    return pl.kernel(
        _agg_body,
        out_type=jax.ShapeDtypeStruct((NC, NP, H), F32),
        mesh=plsc.VectorSubcoreMesh(**_SC_MESH),
        scratch_types=[
            pltpu.VMEM((CH,), jnp.int32),
            pltpu.VMEM((CH,), jnp.int32),
            pltpu.VMEM((CH,), jnp.int32),
            pltpu.VMEM((CH,), jnp.int32),
            pltpu.VMEM((TAIL,), jnp.int32),
            pltpu.VMEM((TAIL,), jnp.int32),
            pltpu.VMEM((CH, H), F32),
            pltpu.VMEM((CH, H), F32),
            pltpu.VMEM((TAIL, H), F32),
            pltpu.VMEM_SHARED((NP, H), F32),
            pltpu.SemaphoreType.DMA,
            pltpu.SemaphoreType.DMA,
        ],
    )


BR = 2000


DEG_OUT_COL = 0
DEG_IN_COL = 64


def _norm_col(d0, d1, col):
    deg = d0[:, col:col + 1] + d1[:, col:col + 1]
    return jax.lax.rsqrt(jnp.maximum(deg, 1.0))


def _proj_body(x_ref, w_ref, b_ref, d0_ref, d1_ref, o_ref):
    nout = _norm_col(d0_ref[...], d1_ref[...], DEG_OUT_COL)
    h = jnp.dot(x_ref[...], w_ref[...], preferred_element_type=F32,
                precision=PREC) + b_ref[...]
    o_ref[...] = h * nout


def _conv_mid_body(a0_ref, a1_ref, d0_ref, d1_ref, w_ref, b_ref, o_ref):
    nin = _norm_col(d0_ref[...], d1_ref[...], DEG_IN_COL)
    agg = (a0_ref[...] + a1_ref[...]) * nin
    h = jnp.dot(agg, w_ref[...], preferred_element_type=F32,
                precision=PREC) + b_ref[...]
    h = jnp.maximum(h, 0.0)
    nout = _norm_col(d0_ref[...], d1_ref[...], DEG_OUT_COL)
    o_ref[...] = h * nout


def _conv_last_body(a0_ref, a1_ref, d0_ref, d1_ref, w_ref, b_ref, o_ref):
    nin = _norm_col(d0_ref[...], d1_ref[...], DEG_IN_COL)
    agg = (a0_ref[...] + a1_ref[...]) * nin
    h = jnp.dot(agg, w_ref[...], preferred_element_type=F32,
                precision=PREC) + b_ref[...]
    o_ref[...] = jnp.maximum(h, 0.0)


def _row_spec(width):
    return pl.BlockSpec((BR, width), lambda i: (i, 0))


def _full_spec(shape):
    return pl.BlockSpec(shape, lambda i: tuple(0 for _ in shape))


def _proj_call(x, w, b, d0, d1):
    return pl.pallas_call(
        _proj_body,
        grid=(N // BR,),
        in_specs=[_row_spec(H), _full_spec((H, H)), _full_spec((1, H)),
                  _row_spec(H), _row_spec(H)],
        out_specs=_row_spec(H),
        out_shape=jax.ShapeDtypeStruct((N, H), F32),
    )(x, w, b, d0, d1)


def _conv_mid_call(a0, a1, d0, d1, w, b):
    return pl.pallas_call(
        _conv_mid_body,
        grid=(N // BR,),
        in_specs=[_row_spec(H), _row_spec(H), _row_spec(H), _row_spec(H),
                  _full_spec((H, H)), _full_spec((1, H))],
        out_specs=_row_spec(H),
        out_shape=jax.ShapeDtypeStruct((N, H), F32),
    )(a0, a1, d0, d1, w, b)


def _conv_last_call(a0, a1, d0, d1, w, b):
    return pl.pallas_call(
        _conv_last_body,
        grid=(N // BR,),
        in_specs=[_row_spec(H), _row_spec(H), _row_spec(H), _row_spec(H),
                  _full_spec((H, H)), _full_spec((1, H))],
        out_specs=_row_spec(H),
        out_shape=jax.ShapeDtypeStruct((N, H), F32),
    )(a0, a1, d0, d1, w, b)


TB = 1000


def _sigmoid(z):
    return 0.5 * jnp.tanh(0.5 * z) + 0.5


def _lstm_body(h2_ref, wih_ref, bih_ref, whh_ref, fcw_ref, fcb_ref, o_ref,
               h_st, c_st, x_st, hblk_st):
    @pl.when(pl.program_id(0) == 0)
    def _init():
        h_st[...] = jnp.zeros_like(h_st)
        c_st[...] = jnp.zeros_like(c_st)

    x_st[...] = jnp.dot(h2_ref[...], wih_ref[...], preferred_element_type=F32,
                        precision=PREC) + bih_ref[...]
    wh = whh_ref[...]

    def step(t, carry):
        h, c = carry
        x = x_st[pl.ds(t, 1), :]
        gates = x + jnp.dot(h.astype(jnp.bfloat16), wh,
                            preferred_element_type=F32, precision=PREC_STEP)
        i = _sigmoid(gates[:, 0:H])
        f = _sigmoid(gates[:, H:2 * H])
        g = jnp.tanh(gates[:, 2 * H:3 * H])
        o = _sigmoid(gates[:, 3 * H:4 * H])
        c = f * c + i * g
        h = o * jnp.tanh(c)
        hblk_st[pl.ds(t, 1), :] = h
        return (h, c)

    hn, cn = lax.fori_loop(0, TB, step, (h_st[...], c_st[...]), unroll=2)
    h_st[...] = hn
    c_st[...] = cn
    o_ref[...] = jnp.dot(hblk_st[...], fcw_ref[...], preferred_element_type=F32,
                         precision=PREC) + fcb_ref[...]


def _lstm_call(h2, wih_t, bias, whh_t, fcw, fcb):
    return pl.pallas_call(
        _lstm_body,
        grid=(N // TB,),
        in_specs=[pl.BlockSpec((TB, H), lambda i: (i, 0)),
                  _full_spec((H, G4)), _full_spec((1, G4)),
                  _full_spec((H, G4)), _full_spec((H, C)), _full_spec((1, C))],
        out_specs=pl.BlockSpec((TB, C), lambda i: (i, 0)),
        out_shape=jax.ShapeDtypeStruct((N, C), F32),
        scratch_shapes=[
            pltpu.VMEM((1, H), F32),
            pltpu.VMEM((1, H), F32),
            pltpu.VMEM((TB, G4), F32),
            pltpu.VMEM((TB, H), F32),
        ],
        compiler_params=pltpu.CompilerParams(
            dimension_semantics=("arbitrary",)),
    )(h2, wih_t, bias, whh_t, fcw, fcb)


def kernel(features, edge_index, proj_W, proj_b, gnn_W0, gnn_b0, gnn_W1,
           gnn_b1, W_ih, W_hh, b_ih, b_hh, fc_W, fc_b):
    src = edge_index[0]
    dst = edge_index[1]
    z128 = jnp.zeros((NP, H), F32)
    col = jnp.arange(H)
    ones_a = jnp.broadcast_to((col < 64).astype(F32), (CH, H))
    ones_b = jnp.broadcast_to((col >= 64).astype(F32), (CH, H))

    deg_p = _get_deg_call()(src, dst, ones_a, ones_b, z128)
    d0, d1 = deg_p[0], deg_p[1]

    hs0 = _proj_call(features, proj_W, proj_b.reshape(1, H), d0, d1)
    agg0 = _get_agg_call()(hs0, src, dst, z128)
    hs1 = _conv_mid_call(agg0[0], agg0[1], d0, d1, gnn_W0,
                         gnn_b0.reshape(1, H))
    agg1 = _get_agg_call()(hs1, src, dst, z128)
    h2 = _conv_last_call(agg1[0], agg1[1], d0, d1, gnn_W1,
                         gnn_b1.reshape(1, H))

    out = _lstm_call(h2, W_ih.T, (b_ih + b_hh).reshape(1, G4),
                     W_hh.T.astype(jnp.bfloat16), fc_W, fc_b.reshape(1, C))
    return out


# speedup vs baseline: 12.7897x; 1.0012x over previous
import functools

import jax
import jax.numpy as jnp
from jax import lax
from jax.experimental import pallas as pl
from jax.experimental.pallas import tpu as pltpu
from jax.experimental.pallas import tpu_sc as plsc

N = 10000
E = 320000
H = 128
C = 16
G4 = 4 * H

NC = 2
NS = 16
NW = NC * NS

NP = 10240
RP = NP // NS
EPW = E // NW
CH = 128
NCHUNK = EPW // CH
TAIL = EPW - NCHUNK * CH

F32 = jnp.float32

_SC_MESH = dict(core_axis_name="c", subcore_axis_name="s",
                num_cores=NC, num_subcores=NS)

PREC = jax.lax.Precision.HIGHEST
PREC_STEP = jax.lax.Precision.DEFAULT


def _deg_body(src_hbm, dst_hbm, ones_a_hbm, ones_b_hbm, z_hbm, out_hbm,
              sidx_v, didx_v, sidx_t, didx_t, ones_a_v, ones_b_v, acc_sh):
    c = lax.axis_index("c")
    s = lax.axis_index("s")
    w = s * NC + c
    rp0 = s * RP
    pltpu.sync_copy(z_hbm.at[pl.ds(rp0, RP), :], acc_sh.at[pl.ds(rp0, RP), :])
    pltpu.sync_copy(ones_a_hbm, ones_a_v)
    pltpu.sync_copy(ones_b_hbm, ones_b_v)
    plsc.subcore_barrier()

    base = w * EPW

    def body(ci, carry):
        off = base + ci * CH
        pltpu.sync_copy(src_hbm.at[pl.ds(off, CH)], sidx_v)
        pltpu.sync_copy(ones_a_v, acc_sh.at[sidx_v], add=True)
        pltpu.sync_copy(dst_hbm.at[pl.ds(off, CH)], didx_v)
        pltpu.sync_copy(ones_b_v, acc_sh.at[didx_v], add=True)
        return carry

    lax.fori_loop(0, NCHUNK, body, 0)

    offt = base + NCHUNK * CH
    pltpu.sync_copy(src_hbm.at[pl.ds(offt, TAIL)], sidx_t)
    pltpu.sync_copy(ones_a_v.at[pl.ds(0, TAIL), :], acc_sh.at[sidx_t], add=True)
    pltpu.sync_copy(dst_hbm.at[pl.ds(offt, TAIL)], didx_t)
    pltpu.sync_copy(ones_b_v.at[pl.ds(0, TAIL), :], acc_sh.at[didx_t], add=True)

    plsc.subcore_barrier()
    pltpu.sync_copy(acc_sh.at[pl.ds(rp0, RP), :], out_hbm.at[c, pl.ds(rp0, RP), :])


@functools.lru_cache(maxsize=None)
def _get_deg_call():
    return pl.kernel(
        _deg_body,
        out_type=jax.ShapeDtypeStruct((NC, NP, H), F32),
        mesh=plsc.VectorSubcoreMesh(**_SC_MESH),
        scratch_types=[
            pltpu.VMEM((CH,), jnp.int32),
            pltpu.VMEM((CH,), jnp.int32),
            pltpu.VMEM((TAIL,), jnp.int32),
            pltpu.VMEM((TAIL,), jnp.int32),
            pltpu.VMEM((CH, H), F32),
            pltpu.VMEM((CH, H), F32),
            pltpu.VMEM_SHARED((NP, H), F32),
        ],
    )


def _agg_body(hs_hbm, src_hbm, dst_hbm, z_hbm, out_hbm,
              sidx0, sidx1, didx0, didx1, sidx_t, didx_t,
              rows0, rows1, rows_t, acc_sh, sem0, sem1):
    c = lax.axis_index("c")
    s = lax.axis_index("s")
    w = s * NC + c
    rp0 = s * RP
    pltpu.sync_copy(z_hbm.at[pl.ds(rp0, RP), :], acc_sh.at[pl.ds(rp0, RP), :])
    plsc.subcore_barrier()

    base = w * EPW

    def gather(idx_ref, rows_ref, sem, off):
        pltpu.sync_copy(src_hbm.at[pl.ds(off, CH)], idx_ref)
        pltpu.async_copy(hs_hbm.at[idx_ref], rows_ref, sem)

    def drain_scatter(idx_ref, rows_ref, sem, off):
        pltpu.make_async_copy(hs_hbm.at[idx_ref], rows_ref, sem).wait()
        pltpu.sync_copy(dst_hbm.at[pl.ds(off, CH)],
                        didx0 if rows_ref is rows0 else didx1)
        pltpu.sync_copy(rows_ref,
                        acc_sh.at[didx0 if rows_ref is rows0 else didx1],
                        add=True)

    gather(sidx0, rows0, sem0, base)

    def pair(k, carry):
        off0 = base + (2 * k) * CH
        gather(sidx1, rows1, sem1, off0 + CH)
        drain_scatter(sidx0, rows0, sem0, off0)
        gather(sidx0, rows0, sem0, off0 + 2 * CH)
        drain_scatter(sidx1, rows1, sem1, off0 + CH)
        return carry

    lax.fori_loop(0, (NCHUNK - 2) // 2, pair, 0)

    off76 = base + (NCHUNK - 2) * CH
    gather(sidx1, rows1, sem1, off76 + CH)
    drain_scatter(sidx0, rows0, sem0, off76)
    offt = base + NCHUNK * CH
    pltpu.sync_copy(src_hbm.at[pl.ds(offt, TAIL)], sidx_t)
    pltpu.async_copy(hs_hbm.at[sidx_t], rows_t, sem0)
    drain_scatter(sidx1, rows1, sem1, off76 + CH)
    pltpu.make_async_copy(hs_hbm.at[sidx_t], rows_t, sem0).wait()
    pltpu.sync_copy(dst_hbm.at[pl.ds(offt, TAIL)], didx_t)
    pltpu.sync_copy(rows_t, acc_sh.at[didx_t], add=True)

    plsc.subcore_barrier()
    pltpu.sync_copy(acc_sh.at[pl.ds(rp0, RP), :], out_hbm.at[c, pl.ds(rp0, RP), :])


@functools.lru_cache(maxsize=None)
def _get_agg_call():
    return pl.kernel(
        _agg_body,
        out_type=jax.ShapeDtypeStruct((NC, NP, H), F32),
        mesh=plsc.VectorSubcoreMesh(**_SC_MESH),
        scratch_types=[
            pltpu.VMEM((CH,), jnp.int32),
            pltpu.VMEM((CH,), jnp.int32),
            pltpu.VMEM((CH,), jnp.int32),
            pltpu.VMEM((CH,), jnp.int32),
            pltpu.VMEM((TAIL,), jnp.int32),
            pltpu.VMEM((TAIL,), jnp.int32),
            pltpu.VMEM((CH, H), F32),
            pltpu.VMEM((CH, H), F32),
            pltpu.VMEM((TAIL, H), F32),
            pltpu.VMEM_SHARED((NP, H), F32),
            pltpu.SemaphoreType.DMA,
            pltpu.SemaphoreType.DMA,
        ],
    )


BR = 2000


DEG_OUT_COL = 0
DEG_IN_COL = 64


def _norm_col(d0, d1, col):
    deg = d0[:, col:col + 1] + d1[:, col:col + 1]
    return jax.lax.rsqrt(jnp.maximum(deg, 1.0))


def _proj_body(x_ref, w_ref, b_ref, d0_ref, d1_ref, o_ref):
    nout = _norm_col(d0_ref[...], d1_ref[...], DEG_OUT_COL)
    h = jnp.dot(x_ref[...], w_ref[...], preferred_element_type=F32,
                precision=PREC) + b_ref[...]
    o_ref[...] = h * nout


def _conv_mid_body(a0_ref, a1_ref, d0_ref, d1_ref, w_ref, b_ref, o_ref):
    nin = _norm_col(d0_ref[...], d1_ref[...], DEG_IN_COL)
    agg = (a0_ref[...] + a1_ref[...]) * nin
    h = jnp.dot(agg, w_ref[...], preferred_element_type=F32,
                precision=PREC) + b_ref[...]
    h = jnp.maximum(h, 0.0)
    nout = _norm_col(d0_ref[...], d1_ref[...], DEG_OUT_COL)
    o_ref[...] = h * nout


def _conv_last_body(a0_ref, a1_ref, d0_ref, d1_ref, w_ref, b_ref, o_ref):
    nin = _norm_col(d0_ref[...], d1_ref[...], DEG_IN_COL)
    agg = (a0_ref[...] + a1_ref[...]) * nin
    h = jnp.dot(agg, w_ref[...], preferred_element_type=F32,
                precision=PREC) + b_ref[...]
    o_ref[...] = jnp.maximum(h, 0.0)


def _row_spec(width):
    return pl.BlockSpec((BR, width), lambda i: (i, 0))


def _full_spec(shape):
    return pl.BlockSpec(shape, lambda i: tuple(0 for _ in shape))


def _proj_call(x, w, b, d0, d1):
    return pl.pallas_call(
        _proj_body,
        grid=(N // BR,),
        in_specs=[_row_spec(H), _full_spec((H, H)), _full_spec((1, H)),
                  _row_spec(H), _row_spec(H)],
        out_specs=_row_spec(H),
        out_shape=jax.ShapeDtypeStruct((N, H), F32),
    )(x, w, b, d0, d1)


def _conv_mid_call(a0, a1, d0, d1, w, b):
    return pl.pallas_call(
        _conv_mid_body,
        grid=(N // BR,),
        in_specs=[_row_spec(H), _row_spec(H), _row_spec(H), _row_spec(H),
                  _full_spec((H, H)), _full_spec((1, H))],
        out_specs=_row_spec(H),
        out_shape=jax.ShapeDtypeStruct((N, H), F32),
    )(a0, a1, d0, d1, w, b)


def _conv_last_call(a0, a1, d0, d1, w, b):
    return pl.pallas_call(
        _conv_last_body,
        grid=(N // BR,),
        in_specs=[_row_spec(H), _row_spec(H), _row_spec(H), _row_spec(H),
                  _full_spec((H, H)), _full_spec((1, H))],
        out_specs=_row_spec(H),
        out_shape=jax.ShapeDtypeStruct((N, H), F32),
    )(a0, a1, d0, d1, w, b)


TB = 1000


def _sigmoid(z):
    return 0.5 * jnp.tanh(0.5 * z) + 0.5


def _lstm_body(a0_ref, a1_ref, d0_ref, d1_ref, w1_ref, b1_ref,
               wih_ref, bih_ref, whh_ref, fcw_ref, fcb_ref, o_ref,
               h_st, c_st, x_st, hblk_st):
    @pl.when(pl.program_id(0) == 0)
    def _init():
        h_st[...] = jnp.zeros_like(h_st)
        c_st[...] = jnp.zeros_like(c_st)

    nin = _norm_col(d0_ref[...], d1_ref[...], DEG_IN_COL)
    agg = (a0_ref[...] + a1_ref[...]) * nin
    h2 = jnp.maximum(jnp.dot(agg, w1_ref[...], preferred_element_type=F32,
                             precision=PREC) + b1_ref[...], 0.0)
    x_st[...] = jnp.dot(h2, wih_ref[...], preferred_element_type=F32,
                        precision=PREC) + bih_ref[...]
    wh = whh_ref[...]

    def step(t, carry):
        h, c = carry
        x = x_st[pl.ds(t, 1), :]
        gates = x + jnp.dot(h.astype(jnp.bfloat16), wh,
                            preferred_element_type=F32, precision=PREC_STEP)
        i = _sigmoid(gates[:, 0:H])
        f = _sigmoid(gates[:, H:2 * H])
        g = jnp.tanh(gates[:, 2 * H:3 * H])
        o = _sigmoid(gates[:, 3 * H:4 * H])
        c = f * c + i * g
        h = o * jnp.tanh(c)
        hblk_st[pl.ds(t, 1), :] = h
        return (h, c)

    hn, cn = lax.fori_loop(0, TB, step, (h_st[...], c_st[...]), unroll=2)
    h_st[...] = hn
    c_st[...] = cn
    o_ref[...] = jnp.dot(hblk_st[...], fcw_ref[...], preferred_element_type=F32,
                         precision=PREC) + fcb_ref[...]


def _lstm_call(a0, a1, d0, d1, w1, b1, wih_t, bias, whh_t, fcw, fcb):
    tspec = pl.BlockSpec((TB, H), lambda i: (i, 0))
    return pl.pallas_call(
        _lstm_body,
        grid=(N // TB,),
        in_specs=[tspec, tspec, tspec, tspec,
                  _full_spec((H, H)), _full_spec((1, H)),
                  _full_spec((H, G4)), _full_spec((1, G4)),
                  _full_spec((H, G4)), _full_spec((H, C)), _full_spec((1, C))],
        out_specs=pl.BlockSpec((TB, C), lambda i: (i, 0)),
        out_shape=jax.ShapeDtypeStruct((N, C), F32),
        scratch_shapes=[
            pltpu.VMEM((1, H), F32),
            pltpu.VMEM((1, H), F32),
            pltpu.VMEM((TB, G4), F32),
            pltpu.VMEM((TB, H), F32),
        ],
        compiler_params=pltpu.CompilerParams(
            dimension_semantics=("arbitrary",)),
    )(a0, a1, d0, d1, w1, b1, wih_t, bias, whh_t, fcw, fcb)


def kernel(features, edge_index, proj_W, proj_b, gnn_W0, gnn_b0, gnn_W1,
           gnn_b1, W_ih, W_hh, b_ih, b_hh, fc_W, fc_b):
    src = edge_index[0]
    dst = edge_index[1]
    z128 = jnp.zeros((NP, H), F32)
    col = jnp.arange(H)
    ones_a = jnp.broadcast_to((col < 64).astype(F32), (CH, H))
    ones_b = jnp.broadcast_to((col >= 64).astype(F32), (CH, H))

    deg_p = _get_deg_call()(src, dst, ones_a, ones_b, z128)
    d0, d1 = deg_p[0], deg_p[1]

    hs0 = _proj_call(features, proj_W, proj_b.reshape(1, H), d0, d1)
    agg0 = _get_agg_call()(hs0, src, dst, z128)
    hs1 = _conv_mid_call(agg0[0], agg0[1], d0, d1, gnn_W0,
                         gnn_b0.reshape(1, H))
    agg1 = _get_agg_call()(hs1, src, dst, z128)

    out = _lstm_call(agg1[0], agg1[1], d0, d1, gnn_W1, gnn_b1.reshape(1, H),
                     W_ih.T, (b_ih + b_hh).reshape(1, G4),
                     W_hh.T.astype(jnp.bfloat16), fc_W, fc_b.reshape(1, C))
    return out
